# Initial kernel scaffold; baseline (speedup 1.0000x reference)
import functools

import jax
import jax.numpy as jnp
from jax import lax
from jax.experimental import pallas as pl
from jax.experimental.pallas import tpu as pltpu
from jax.experimental.pallas import tpu_sc as plsc

_N = 10000
_E = 320000
_D = 128
_K = 5
_DIM = 3
_EPS = 1e-15

_NC, _NS, _L = 2, 16, 16
_NW = _NC * _NS
_EW = _E // _NW
_C = 80
_NCHUNK = _EW // _C
_RPT = _N // _NS

_f32 = jnp.float32
_i32 = jnp.int32


def _gauss_body(ea_ref, mu1_ref, sg1_ref, mu2_ref, sg2_ref, mus_ref,
                sgs_ref, g1_ref, g2_ref, gs_ref):
    ea = ea_ref[...]

    def gm(mu, sg):
        inv = -0.5 / (_EPS + sg * sg)
        diff = ea[:, None, :] - mu[None, :, :]
        return jnp.exp(jnp.sum(diff * diff * inv[None, :, :], axis=-1))

    g1_ref[...] = gm(mu1_ref[...], sg1_ref[...])
    g2_ref[...] = gm(mu2_ref[...], sg2_ref[...])
    gs_ref[...] = gm(mus_ref[...], sgs_ref[...])


def _gauss_weights(edge_attr, mu1, sigma1, mu2, sigma2, mus, sigmas):
    eb = 8000
    grid = _E // eb
    full = lambda s: pl.BlockSpec(s, lambda i: (0, 0))
    return pl.pallas_call(
        _gauss_body,
        grid=(grid,),
        in_specs=[pl.BlockSpec((eb, _DIM), lambda i: (i, 0)),
                  full((_K, _DIM)), full((_K, _DIM)),
                  full((_K, _DIM)), full((_K, _DIM)),
                  full((1, _DIM)), full((1, _DIM))],
        out_specs=[pl.BlockSpec((eb, _K), lambda i: (i, 0)),
                   pl.BlockSpec((eb, _K), lambda i: (i, 0)),
                   pl.BlockSpec((eb, 1), lambda i: (i, 0))],
        out_shape=[jax.ShapeDtypeStruct((_E, _K), _f32),
                   jax.ShapeDtypeStruct((_E, _K), _f32),
                   jax.ShapeDtypeStruct((_E, 1), _f32)],
    )(edge_attr, mu1, sigma1, mu2, sigma2, mus, sigmas)


def _dense_body(nout, x_ref, *refs):
    x = x_ref[...]
    for wi in range(nout):
        refs[nout + wi][...] = lax.dot_general(
            x, refs[wi][...], (((1,), (0,)), ((), ())),
            preferred_element_type=_f32)


def _dense(x, weights):
    nb = 2500
    grid = x.shape[0] // nb
    din = x.shape[1]
    in_specs = [pl.BlockSpec((nb, din), lambda i: (i, 0))]
    out_specs, out_shape = [], []
    for w in weights:
        dout = w.shape[1]
        in_specs.append(pl.BlockSpec((din, dout), lambda i: (0, 0)))
        out_specs.append(pl.BlockSpec((nb, dout), lambda i: (i, 0)))
        out_shape.append(jax.ShapeDtypeStruct((x.shape[0], dout), _f32))
    return pl.pallas_call(
        functools.partial(_dense_body, len(weights)),
        grid=(grid,),
        in_specs=in_specs, out_specs=out_specs, out_shape=out_shape,
    )(x, *weights)


def _bn(y, gamma, beta):
    m = jnp.mean(y, axis=0)
    v = jnp.mean((y - m) ** 2, axis=0)
    return gamma * (y - m) / jnp.sqrt(v + 1e-5) + beta


def _elu(y):
    return jnp.where(y > 0, y, jnp.expm1(y))


def _mid_body(acc_ref, xr_ref, b_ref, gam_ref, bet_ref, h_ref, cnt_ref):
    s = acc_ref[0:_N, 0:_D] + acc_ref[_N:2 * _N, 0:_D]
    c = acc_ref[0:_N, _D:_D + 1] + acc_ref[_N:2 * _N, _D:_D + 1]
    cnt = jnp.maximum(c, 1.0)
    y = s / cnt + xr_ref[...] + b_ref[...][None, :]
    h_ref[...] = _elu(_bn(y, gam_ref[...][None, :], bet_ref[...][None, :]))
    cnt_ref[...] = cnt


def _mid(acc, xr, bias, gamma, beta):
    return pl.pallas_call(
        _mid_body,
        out_shape=[jax.ShapeDtypeStruct((_N, _D), _f32),
                   jax.ShapeDtypeStruct((_N, 1), _f32)],
    )(acc, xr, bias, gamma, beta)


def _final_body(acc2_ref, accs_ref, cnt_ref, xr2_ref, b2_ref, gam2_ref,
                bet2_ref, xrs_ref, bs_ref, gams_ref, bets_ref, o_ref):
    cnt = cnt_ref[...]
    y2 = ((acc2_ref[0:_N, :] + acc2_ref[_N:2 * _N, :]) / cnt
          + xr2_ref[...] + b2_ref[...][None, :])
    h = _bn(y2, gam2_ref[...][None, :], bet2_ref[...][None, :])
    ys = ((accs_ref[0:_N, :] + accs_ref[_N:2 * _N, :]) / cnt
          + xrs_ref[...] + bs_ref[...][None, :])
    sc = _bn(ys, gams_ref[...][None, :], bets_ref[...][None, :])
    o_ref[...] = _elu(h + sc)


def _final(acc2, accs, cnt, xr2, b2, gam2, bet2, xrs, bs, gams, bets):
    return pl.pallas_call(
        _final_body,
        out_shape=jax.ShapeDtypeStruct((_N, _D), _f32),
    )(acc2, accs, cnt, xr2, b2, gam2, bet2, xrs, bs, gams, bets)


def _sc_agg_body(k_loc, w, table, srcs, dsts, gss, zeros_in, out,
                 src_v, dst_v, gss_v, rows_v, msg_v, acc_sh, sem):
    cid = lax.axis_index("c")
    sid = lax.axis_index("s")
    wid = cid * _NS + sid

    r0 = sid * _RPT
    pltpu.sync_copy(zeros_in.at[pl.ds(r0, _RPT)], acc_sh.at[pl.ds(r0, _RPT)])

    iota16 = lax.iota(_i32, 16)
    if w > _D:
        ones16 = jnp.ones((16,), _f32)

        def ones_init(r, carry):
            plsc.store_scatter(msg_v, [jnp.full((16,), r, _i32), iota16 + _D],
                               ones16)
            return carry
        lax.fori_loop(0, _C, ones_init, 0)

    plsc.subcore_barrier()

    msg_cols = [iota16 + (j8 * 16) for j8 in range(_D // 16)]

    def chunk(i, carry):
        base = wid * _EW + i * _C
        pltpu.sync_copy(srcs.at[pl.ds(base, _C)], src_v)
        pltpu.sync_copy(dsts.at[pl.ds(base, _C)], dst_v)
        pltpu.sync_copy(gss.at[pl.ds(base * k_loc, _C * k_loc)], gss_v)
        pltpu.async_copy(table.at[src_v], rows_v, sem).wait()

        def group(g, c2):
            for el in range(_L):
                e = g * _L + el
                row_ix = jnp.full((16,), e, _i32)
                gb = [plsc.load_gather(
                          gss_v, [jnp.full((16,), e * k_loc + k, _i32)])
                      for k in range(k_loc)]
                for j8 in range(_D // 16):
                    acc = gb[0] * plsc.load_gather(
                        rows_v, [row_ix, msg_cols[j8]])
                    for k in range(1, k_loc):
                        acc = acc + gb[k] * plsc.load_gather(
                            rows_v, [row_ix, msg_cols[j8] + k * _D])
                    plsc.store_scatter(msg_v, [row_ix, msg_cols[j8]], acc)
            return c2
        lax.fori_loop(0, _C // _L, group, 0)

        pltpu.sync_copy(msg_v, acc_sh.at[dst_v], add=True)
        return carry

    lax.fori_loop(0, _NCHUNK, chunk, 0)
    plsc.subcore_barrier()

    pltpu.sync_copy(acc_sh.at[pl.ds(r0, _RPT)],
                    out.at[pl.ds(cid * _N + r0, _RPT)])


def _sc_agg(table, srcs, dsts, gss_flat, with_cnt):
    k_loc = table.shape[1] // _D
    w = _D + 16 if with_cnt else _D
    zeros_in = jnp.zeros((_N, w), _f32)
    mesh = plsc.VectorSubcoreMesh(core_axis_name="c", subcore_axis_name="s")
    kfn = pl.kernel(
        functools.partial(_sc_agg_body, k_loc, w),
        out_type=jax.ShapeDtypeStruct((2 * _N, w), _f32),
        mesh=mesh,
        scratch_types=[
            pltpu.VMEM((_C,), _i32),
            pltpu.VMEM((_C,), _i32),
            pltpu.VMEM((_C * k_loc,), _f32),
            pltpu.VMEM((_C, k_loc * _D), _f32),
            pltpu.VMEM((_C, w), _f32),
            pltpu.VMEM_SHARED((_N, w), _f32),
            pltpu.SemaphoreType.DMA,
        ],
    )
    return kfn(table, srcs, dsts, gss_flat, zeros_in)


def kernel(x, edge_index, edge_attr, g1, mu1, sigma1, root1, bias1, gamma1,
           beta1, g2, mu2, sigma2, root2, bias2, gamma2, beta2, gs, mus,
           sigmas, roots, biass, gammas, betas):
    src = edge_index[0]
    dst = edge_index[1]

    gauss1, gauss2, gausss = _gauss_weights(
        edge_attr, mu1, sigma1, mu2, sigma2, mus, sigmas)
    gauss1f = gauss1.reshape(-1)
    gauss2f = gauss2.reshape(-1)
    gausssf = gausss.reshape(-1)

    xg1, xr1, xgs, xrs = _dense(x, [g1, root1, gs, roots])

    acc1 = _sc_agg(xg1, src, dst, gauss1f, with_cnt=True)
    h, cnt = _mid(acc1, xr1, bias1, gamma1, beta1)

    xg2, xr2 = _dense(h, [g2, root2])
    acc2 = _sc_agg(xg2, src, dst, gauss2f, with_cnt=False)
    accs = _sc_agg(xgs, src, dst, gausssf, with_cnt=False)

    return _final(acc2, accs, cnt, xr2, bias2, gamma2, beta2,
                  xrs, biass, gammas, betas)


# speedup vs baseline: 1.1247x; 1.1247x over previous
import functools

import jax
import jax.numpy as jnp
from jax import lax
from jax.experimental import pallas as pl
from jax.experimental.pallas import tpu as pltpu
from jax.experimental.pallas import tpu_sc as plsc

_N = 10000
_E = 320000
_D = 128
_K = 5
_DIM = 3
_EPS = 1e-15

_NC, _NS, _L = 2, 16, 16
_NW = _NC * _NS
_EW = _E // _NW
_C = 40
_NCHUNK = _EW // _C
_RPT = _N // _NS

_f32 = jnp.float32
_i32 = jnp.int32


def _gauss_body(ea_ref, mu1_ref, sg1_ref, mu2_ref, sg2_ref, mus_ref,
                sgs_ref, g1_ref, g2_ref, gs_ref):
    ea = ea_ref[...]

    def gm(mu, sg):
        inv = -0.5 / (_EPS + sg * sg)
        diff = ea[:, None, :] - mu[None, :, :]
        return jnp.exp(jnp.sum(diff * diff * inv[None, :, :], axis=-1))

    g1_ref[...] = gm(mu1_ref[...], sg1_ref[...])
    g2_ref[...] = gm(mu2_ref[...], sg2_ref[...])
    gs_ref[...] = gm(mus_ref[...], sgs_ref[...])


def _gauss_weights(edge_attr, mu1, sigma1, mu2, sigma2, mus, sigmas):
    eb = 2000
    grid = _E // eb
    full = lambda s: pl.BlockSpec(s, lambda i: (0, 0))
    return pl.pallas_call(
        _gauss_body,
        grid=(grid,),
        in_specs=[pl.BlockSpec((eb, _DIM), lambda i: (i, 0)),
                  full((_K, _DIM)), full((_K, _DIM)),
                  full((_K, _DIM)), full((_K, _DIM)),
                  full((1, _DIM)), full((1, _DIM))],
        out_specs=[pl.BlockSpec((eb, _K), lambda i: (i, 0)),
                   pl.BlockSpec((eb, _K), lambda i: (i, 0)),
                   pl.BlockSpec((eb, 1), lambda i: (i, 0))],
        out_shape=[jax.ShapeDtypeStruct((_E, _K), _f32),
                   jax.ShapeDtypeStruct((_E, _K), _f32),
                   jax.ShapeDtypeStruct((_E, 1), _f32)],
    )(edge_attr, mu1, sigma1, mu2, sigma2, mus, sigmas)


def _dense_body(nout, x_ref, *refs):
    x = x_ref[...]
    for wi in range(nout):
        refs[nout + wi][...] = lax.dot_general(
            x, refs[wi][...], (((1,), (0,)), ((), ())),
            preferred_element_type=_f32)


def _dense(x, weights):
    nb = 2000
    grid = x.shape[0] // nb
    din = x.shape[1]
    in_specs = [pl.BlockSpec((nb, din), lambda i: (i, 0))]
    out_specs, out_shape = [], []
    for w in weights:
        dout = w.shape[1]
        in_specs.append(pl.BlockSpec((din, dout), lambda i: (0, 0)))
        out_specs.append(pl.BlockSpec((nb, dout), lambda i: (i, 0)))
        out_shape.append(jax.ShapeDtypeStruct((x.shape[0], dout), _f32))
    return pl.pallas_call(
        functools.partial(_dense_body, len(weights)),
        grid=(grid,),
        in_specs=in_specs, out_specs=out_specs, out_shape=out_shape,
    )(x, *weights)


def _bn(y, gamma, beta):
    m = jnp.mean(y, axis=0)
    v = jnp.mean((y - m) ** 2, axis=0)
    return gamma * (y - m) / jnp.sqrt(v + 1e-5) + beta


def _elu(y):
    return jnp.where(y > 0, y, jnp.exp(jnp.minimum(y, 0.0)) - 1.0)


def _mid_body(acc_ref, xr_ref, b_ref, gam_ref, bet_ref, h_ref, cnt_ref):
    s = acc_ref[0:_N, 0:_D] + acc_ref[_N:2 * _N, 0:_D]
    c = acc_ref[0:_N, _D:_D + 1] + acc_ref[_N:2 * _N, _D:_D + 1]
    cnt = jnp.maximum(c, 1.0)
    y = s / cnt + xr_ref[...] + b_ref[...][None, :]
    h_ref[...] = _elu(_bn(y, gam_ref[...][None, :], bet_ref[...][None, :]))
    cnt_ref[...] = cnt


def _mid(acc, xr, bias, gamma, beta):
    return pl.pallas_call(
        _mid_body,
        out_shape=[jax.ShapeDtypeStruct((_N, _D), _f32),
                   jax.ShapeDtypeStruct((_N, 1), _f32)],
    )(acc, xr, bias, gamma, beta)


def _final_body(acc2_ref, accs_ref, cnt_ref, xr2_ref, b2_ref, gam2_ref,
                bet2_ref, xrs_ref, bs_ref, gams_ref, bets_ref, o_ref):
    cnt = cnt_ref[...]
    y2 = ((acc2_ref[0:_N, :] + acc2_ref[_N:2 * _N, :]) / cnt
          + xr2_ref[...] + b2_ref[...][None, :])
    h = _bn(y2, gam2_ref[...][None, :], bet2_ref[...][None, :])
    ys = ((accs_ref[0:_N, :] + accs_ref[_N:2 * _N, :]) / cnt
          + xrs_ref[...] + bs_ref[...][None, :])
    sc = _bn(ys, gams_ref[...][None, :], bets_ref[...][None, :])
    o_ref[...] = _elu(h + sc)


def _final(acc2, accs, cnt, xr2, b2, gam2, bet2, xrs, bs, gams, bets):
    return pl.pallas_call(
        _final_body,
        out_shape=jax.ShapeDtypeStruct((_N, _D), _f32),
    )(acc2, accs, cnt, xr2, b2, gam2, bet2, xrs, bs, gams, bets)


def _sc_agg_body(k_loc, w, table, srcs, dsts, gss, zeros_in, out,
                 src_v, dst_v, gss_v, rows_v, msg_v, acc_sh, sem):
    cid = lax.axis_index("c")
    sid = lax.axis_index("s")
    wid = cid * _NS + sid

    r0 = sid * _RPT
    pltpu.sync_copy(zeros_in.at[pl.ds(r0, _RPT)], acc_sh.at[pl.ds(r0, _RPT)])

    iota16 = lax.iota(_i32, 16)
    if w > _D:
        ones16 = jnp.ones((16,), _f32)

        def ones_init(r, carry):
            plsc.store_scatter(msg_v, [jnp.full((16,), r, _i32), iota16 + _D],
                               ones16)
            return carry
        lax.fori_loop(0, _C, ones_init, 0)

    plsc.subcore_barrier()

    msg_cols = [iota16 + (j8 * 16) for j8 in range(_D // 16)]

    def chunk(i, carry):
        base = wid * _EW + i * _C
        pltpu.sync_copy(srcs.at[pl.ds(base, _C)], src_v)
        pltpu.sync_copy(dsts.at[pl.ds(base, _C)], dst_v)
        pltpu.sync_copy(gss.at[pl.ds(base * k_loc, _C * k_loc)], gss_v)
        pltpu.async_copy(table.at[src_v], rows_v, sem).wait()

        def do_edge(e):
            row_ix = jnp.full((16,), e, _i32)
            gb = [plsc.load_gather(
                      gss_v, [jnp.full((16,), e * k_loc + k, _i32)])
                  for k in range(k_loc)]
            for j8 in range(_D // 16):
                acc = gb[0] * plsc.load_gather(
                    rows_v, [row_ix, msg_cols[j8]])
                for k in range(1, k_loc):
                    acc = acc + gb[k] * plsc.load_gather(
                        rows_v, [row_ix, msg_cols[j8] + k * _D])
                plsc.store_scatter(msg_v, [row_ix, msg_cols[j8]], acc)

        def group(g, c2):
            for el in range(_L):
                do_edge(g * _L + el)
            return c2
        lax.fori_loop(0, _C // _L, group, 0)
        for el in range(_C % _L):
            do_edge((_C // _L) * _L + el)

        pltpu.sync_copy(msg_v, acc_sh.at[dst_v], add=True)
        return carry

    lax.fori_loop(0, _NCHUNK, chunk, 0)
    plsc.subcore_barrier()

    pltpu.sync_copy(acc_sh.at[pl.ds(r0, _RPT)],
                    out.at[pl.ds(cid * _N + r0, _RPT)])


def _sc_agg(table, srcs, dsts, gss_flat, with_cnt):
    k_loc = table.shape[1] // _D
    w = _D + 16 if with_cnt else _D
    zeros_in = jnp.zeros((_N, w), _f32)
    mesh = plsc.VectorSubcoreMesh(core_axis_name="c", subcore_axis_name="s")
    kfn = pl.kernel(
        functools.partial(_sc_agg_body, k_loc, w),
        out_type=jax.ShapeDtypeStruct((2 * _N, w), _f32),
        mesh=mesh,
        compiler_params=pltpu.CompilerParams(use_tc_tiling_on_sc=False,
                                             needs_layout_passes=False),
        scratch_types=[
            pltpu.VMEM((_C,), _i32),
            pltpu.VMEM((_C,), _i32),
            pltpu.VMEM((_C * k_loc,), _f32),
            pltpu.VMEM((_C, k_loc * _D), _f32),
            pltpu.VMEM((_C, w), _f32),
            pltpu.VMEM_SHARED((_N, w), _f32),
            pltpu.SemaphoreType.DMA,
        ],
    )
    return kfn(table, srcs, dsts, gss_flat, zeros_in)


def kernel(x, edge_index, edge_attr, g1, mu1, sigma1, root1, bias1, gamma1,
           beta1, g2, mu2, sigma2, root2, bias2, gamma2, beta2, gs, mus,
           sigmas, roots, biass, gammas, betas):
    src = edge_index[0]
    dst = edge_index[1]

    gauss1, gauss2, gausss = _gauss_weights(
        edge_attr, mu1, sigma1, mu2, sigma2, mus, sigmas)
    gauss1f = gauss1.reshape(-1)
    gauss2f = gauss2.reshape(-1)
    gausssf = gausss.reshape(-1)

    xg1, xr1, xgs, xrs = _dense(x, [g1, root1, gs, roots])

    acc1 = _sc_agg(xg1, src, dst, gauss1f, with_cnt=True)
    h, cnt = _mid(acc1, xr1, bias1, gamma1, beta1)

    xg2, xr2 = _dense(h, [g2, root2])
    acc2 = _sc_agg(xg2, src, dst, gauss2f, with_cnt=False)
    accs = _sc_agg(xgs, src, dst, gausssf, with_cnt=False)

    return _final(acc2, accs, cnt, xr2, bias2, gamma2, beta2,
                  xrs, biass, gammas, betas)


# speedup vs baseline: 2.6272x; 2.3359x over previous
import functools

import jax
import jax.numpy as jnp
from jax import lax
from jax.experimental import pallas as pl
from jax.experimental.pallas import tpu as pltpu
from jax.experimental.pallas import tpu_sc as plsc

_N = 10000
_E = 320000
_D = 128
_K = 5
_DIM = 3
_EPS = 1e-15

_NC, _NS, _L = 2, 16, 16
_NW = _NC * _NS
_EW = _E // _NW
_C = 40
_NCHUNK = _EW // _C
_RPT = _N // _NS

_f32 = jnp.float32
_i32 = jnp.int32


def _gauss_body(ea_ref, mu1_ref, sg1_ref, mu2_ref, sg2_ref, mus_ref,
                sgs_ref, g1_ref, g2_ref, gs_ref):
    ea = ea_ref[...]

    def gm(mu, sg):
        inv = -0.5 / (_EPS + sg * sg)
        diff = ea[:, None, :] - mu[None, :, :]
        return jnp.exp(jnp.sum(diff * diff * inv[None, :, :], axis=-1))

    g1_ref[...] = gm(mu1_ref[...], sg1_ref[...])
    g2_ref[...] = gm(mu2_ref[...], sg2_ref[...])
    gs_ref[...] = gm(mus_ref[...], sgs_ref[...])


def _gauss_weights(edge_attr, mu1, sigma1, mu2, sigma2, mus, sigmas):
    eb = 2000
    grid = _E // eb
    full = lambda s: pl.BlockSpec(s, lambda i: (0, 0))
    return pl.pallas_call(
        _gauss_body,
        grid=(grid,),
        in_specs=[pl.BlockSpec((eb, _DIM), lambda i: (i, 0)),
                  full((_K, _DIM)), full((_K, _DIM)),
                  full((_K, _DIM)), full((_K, _DIM)),
                  full((1, _DIM)), full((1, _DIM))],
        out_specs=[pl.BlockSpec((eb, _K), lambda i: (i, 0)),
                   pl.BlockSpec((eb, _K), lambda i: (i, 0)),
                   pl.BlockSpec((eb, 1), lambda i: (i, 0))],
        out_shape=[jax.ShapeDtypeStruct((_E, _K), _f32),
                   jax.ShapeDtypeStruct((_E, _K), _f32),
                   jax.ShapeDtypeStruct((_E, 1), _f32)],
    )(edge_attr, mu1, sigma1, mu2, sigma2, mus, sigmas)


def _dense_body(nout, x_ref, *refs):
    x = x_ref[...]
    for wi in range(nout):
        refs[nout + wi][...] = lax.dot_general(
            x, refs[wi][...], (((1,), (0,)), ((), ())),
            preferred_element_type=_f32)


def _dense(x, weights):
    nb = 2000
    grid = x.shape[0] // nb
    din = x.shape[1]
    in_specs = [pl.BlockSpec((nb, din), lambda i: (i, 0))]
    out_specs, out_shape = [], []
    for w in weights:
        dout = w.shape[1]
        in_specs.append(pl.BlockSpec((din, dout), lambda i: (0, 0)))
        out_specs.append(pl.BlockSpec((nb, dout), lambda i: (i, 0)))
        out_shape.append(jax.ShapeDtypeStruct((x.shape[0], dout), _f32))
    return pl.pallas_call(
        functools.partial(_dense_body, len(weights)),
        grid=(grid,),
        in_specs=in_specs, out_specs=out_specs, out_shape=out_shape,
    )(x, *weights)


def _bn(y, gamma, beta):
    m = jnp.mean(y, axis=0)
    v = jnp.mean((y - m) ** 2, axis=0)
    return gamma * (y - m) / jnp.sqrt(v + 1e-5) + beta


def _elu(y):
    return jnp.where(y > 0, y, jnp.exp(jnp.minimum(y, 0.0)) - 1.0)


def _mid_body(acc_ref, accs_ref, xr_ref, b_ref, gam_ref, bet_ref, h_ref,
              cnt_ref):
    s = acc_ref[0:_N, :] + acc_ref[_N:2 * _N, :]
    c = accs_ref[0:_N, _D:_D + 1] + accs_ref[_N:2 * _N, _D:_D + 1]
    cnt = jnp.maximum(c, 1.0)
    y = s / cnt + xr_ref[...] + b_ref[...][None, :]
    h_ref[...] = _elu(_bn(y, gam_ref[...][None, :], bet_ref[...][None, :]))
    cnt_ref[...] = cnt


def _mid(acc, accs, xr, bias, gamma, beta):
    return pl.pallas_call(
        _mid_body,
        out_shape=[jax.ShapeDtypeStruct((_N, _D), _f32),
                   jax.ShapeDtypeStruct((_N, 1), _f32)],
    )(acc, accs, xr, bias, gamma, beta)


def _final_body(acc2_ref, accs_ref, cnt_ref, xr2_ref, b2_ref, gam2_ref,
                bet2_ref, xrs_ref, bs_ref, gams_ref, bets_ref, o_ref):
    cnt = cnt_ref[...]
    y2 = ((acc2_ref[0:_N, :] + acc2_ref[_N:2 * _N, :]) / cnt
          + xr2_ref[...] + b2_ref[...][None, :])
    h = _bn(y2, gam2_ref[...][None, :], bet2_ref[...][None, :])
    ys = ((accs_ref[0:_N, 0:_D] + accs_ref[_N:2 * _N, 0:_D]) / cnt
          + xrs_ref[...] + bs_ref[...][None, :])
    sc = _bn(ys, gams_ref[...][None, :], bets_ref[...][None, :])
    o_ref[...] = _elu(h + sc)


def _final(acc2, accs, cnt, xr2, b2, gam2, bet2, xrs, bs, gams, bets):
    return pl.pallas_call(
        _final_body,
        out_shape=jax.ShapeDtypeStruct((_N, _D), _f32),
        compiler_params=pltpu.CompilerParams(
            vmem_limit_bytes=100 * 1024 * 1024),
    )(acc2, accs, cnt, xr2, b2, gam2, bet2, xrs, bs, gams, bets)


def _pack_bf16(t):
    n, d = t.shape
    tb = t.astype(jnp.bfloat16).reshape(n, d // 2, 2)
    return jax.lax.bitcast_convert_type(tb, _i32)


def _sc_agg_body(k_loc, w, packed, table, srcs, dsts, gss, zeros_in, out,
                 src_v0, src_v1, dst_v0, dst_v1, gss_v0, gss_v1,
                 rows_v0, rows_v1, msg_v, acc_sh,
                 sem_i0, sem_i1, sem_r0, sem_r1):
    src_v = [src_v0, src_v1]
    dst_v = [dst_v0, dst_v1]
    gss_v = [gss_v0, gss_v1]
    rows_v = [rows_v0, rows_v1]
    sem_i = [sem_i0, sem_i1]
    sem_r = [sem_r0, sem_r1]

    cid = lax.axis_index("c")
    sid = lax.axis_index("s")
    wid = cid * _NS + sid

    r0 = sid * _RPT
    pltpu.sync_copy(zeros_in.at[pl.ds(r0, _RPT)], acc_sh.at[pl.ds(r0, _RPT)])

    iota16 = lax.iota(_i32, 16)
    if w > _D:
        ones16 = jnp.ones((16,), _f32)

        def ones_init(r, carry):
            plsc.store_scatter(msg_v, [jnp.full((16,), r, _i32), iota16 + _D],
                               ones16)
            return carry
        lax.fori_loop(0, _C, ones_init, 0)

    plsc.subcore_barrier()

    def issue_idx(j, p):
        base = wid * _EW + j * _C
        pltpu.async_copy(srcs.at[pl.ds(base, _C)], src_v[p], sem_i[p])
        pltpu.async_copy(dsts.at[pl.ds(base, _C)], dst_v[p], sem_i[p])
        pltpu.async_copy(gss.at[pl.ds(base * k_loc, _C * k_loc)],
                         gss_v[p], sem_i[p])

    def wait_idx(p):
        pltpu.make_async_copy(srcs.at[pl.ds(0, _C)], src_v[p],
                              sem_i[p]).wait()
        pltpu.make_async_copy(dsts.at[pl.ds(0, _C)], dst_v[p],
                              sem_i[p]).wait()
        pltpu.make_async_copy(gss.at[pl.ds(0, _C * k_loc)], gss_v[p],
                              sem_i[p]).wait()

    def issue_rows(p):
        pltpu.async_copy(table.at[src_v[p]], rows_v[p], sem_r[p])

    def wait_rows(p):
        pltpu.make_async_copy(table.at[src_v[p]], rows_v[p],
                              sem_r[p]).wait()

    if packed:
        st_cols = ([iota16 * 2 + 32 * b for b in range(_D // 32)]
                   + [iota16 * 2 + 1 + 32 * b for b in range(_D // 32)])
    else:
        msg_cols = [iota16 + (j8 * 16) for j8 in range(_D // 16)]

    def compute_chunk(p):
        def do_edge(e):
            row_ix = jnp.full((16,), e, _i32)
            gb = [plsc.load_gather(
                      gss_v[p], [jnp.full((16,), e * k_loc + k, _i32)])
                  for k in range(k_loc)]
            if packed:
                nb = _D // 32
                acc_e = [None] * nb
                acc_o = [None] * nb
                for k in range(k_loc):
                    for b in range(nb):
                        col = iota16 + (k * (_D // 2) + b * 16)
                        wv = plsc.load_gather(rows_v[p], [row_ix, col])
                        lo, hi = plsc.unpack(
                            plsc.bitcast(wv, jnp.bfloat16),
                            format=plsc.PackFormat.INTERLEAVED)
                        if k == 0:
                            acc_e[b] = gb[0] * lo
                            acc_o[b] = gb[0] * hi
                        else:
                            acc_e[b] = acc_e[b] + gb[k] * lo
                            acc_o[b] = acc_o[b] + gb[k] * hi
                for b in range(nb):
                    plsc.store_scatter(msg_v, [row_ix, st_cols[b]], acc_e[b])
                    plsc.store_scatter(msg_v, [row_ix, st_cols[nb + b]],
                                       acc_o[b])
            else:
                for j8 in range(_D // 16):
                    acc = gb[0] * plsc.load_gather(
                        rows_v[p], [row_ix, msg_cols[j8]])
                    for k in range(1, k_loc):
                        acc = acc + gb[k] * plsc.load_gather(
                            rows_v[p], [row_ix, msg_cols[j8] + k * _D])
                    plsc.store_scatter(msg_v, [row_ix, msg_cols[j8]], acc)

        def group(g, c2):
            for el in range(4):
                do_edge(g * 4 + el)
            return c2
        lax.fori_loop(0, _C // 4, group, 0)

    issue_idx(0, 0)
    issue_idx(1, 1)
    wait_idx(0)
    issue_rows(0)

    def pair(t, carry):
        for p in range(2):
            j = t * 2 + p
            q = 1 - p
            wait_rows(p)
            wait_idx(q)
            issue_rows(q)
            compute_chunk(p)
            pltpu.sync_copy(msg_v, acc_sh.at[dst_v[p]], add=True)
            issue_idx(lax.rem(j + 2, _NCHUNK), p)
        return carry

    lax.fori_loop(0, _NCHUNK // 2, pair, 0)
    wait_rows(0)
    wait_idx(1)
    plsc.subcore_barrier()

    pltpu.sync_copy(acc_sh.at[pl.ds(r0, _RPT)],
                    out.at[pl.ds(cid * _N + r0, _RPT)])


def _sc_agg(table, srcs, dsts, gss_flat, with_cnt):
    packed = table.dtype == _i32
    k_loc = table.shape[1] // (_D // 2 if packed else _D)
    w = _D + 16 if with_cnt else _D
    tw = table.shape[1]
    zeros_in = jnp.zeros((_N, w), _f32)
    tdt = _i32 if packed else _f32
    mesh = plsc.VectorSubcoreMesh(core_axis_name="c", subcore_axis_name="s")
    kfn = pl.kernel(
        functools.partial(_sc_agg_body, k_loc, w, packed),
        out_type=jax.ShapeDtypeStruct((2 * _N, w), _f32),
        mesh=mesh,
        compiler_params=pltpu.CompilerParams(use_tc_tiling_on_sc=False,
                                             needs_layout_passes=False),
        scratch_types=[
            pltpu.VMEM((_C,), _i32),
            pltpu.VMEM((_C,), _i32),
            pltpu.VMEM((_C,), _i32),
            pltpu.VMEM((_C,), _i32),
            pltpu.VMEM((_C * k_loc,), _f32),
            pltpu.VMEM((_C * k_loc,), _f32),
            pltpu.VMEM((_C, tw), tdt),
            pltpu.VMEM((_C, tw), tdt),
            pltpu.VMEM((_C, w), _f32),
            pltpu.VMEM_SHARED((_N, w), _f32),
            pltpu.SemaphoreType.DMA,
            pltpu.SemaphoreType.DMA,
            pltpu.SemaphoreType.DMA,
            pltpu.SemaphoreType.DMA,
        ],
    )
    return kfn(table, srcs, dsts, gss_flat, zeros_in)


def kernel(x, edge_index, edge_attr, g1, mu1, sigma1, root1, bias1, gamma1,
           beta1, g2, mu2, sigma2, root2, bias2, gamma2, beta2, gs, mus,
           sigmas, roots, biass, gammas, betas):
    src = edge_index[0]
    dst = edge_index[1]

    gauss1, gauss2, gausss = _gauss_weights(
        edge_attr, mu1, sigma1, mu2, sigma2, mus, sigmas)
    gauss1f = gauss1.reshape(-1)
    gauss2f = gauss2.reshape(-1)
    gausssf = gausss.reshape(-1)

    xg1, xr1, xgs, xrs = _dense(x, [g1, root1, gs, roots])
    xg1p = _pack_bf16(xg1)

    accs = _sc_agg(xgs, src, dst, gausssf, with_cnt=True)
    acc1 = _sc_agg(xg1p, src, dst, gauss1f, with_cnt=False)
    h, cnt = _mid(acc1, accs, xr1, bias1, gamma1, beta1)

    xg2, xr2 = _dense(h, [g2, root2])
    acc2 = _sc_agg(_pack_bf16(xg2), src, dst, gauss2f, with_cnt=False)

    return _final(acc2, accs, cnt, xr2, bias2, gamma2, beta2,
                  xrs, biass, gammas, betas)


# speedup vs baseline: 4.0809x; 1.5533x over previous
import functools

import jax
import jax.numpy as jnp
from jax import lax
from jax.experimental import pallas as pl
from jax.experimental.pallas import tpu as pltpu
from jax.experimental.pallas import tpu_sc as plsc

_N = 10000
_E = 320000
_D = 128
_K = 5
_DIM = 3
_EPS = 1e-15

_NC, _NS, _L = 2, 16, 16
_NW = _NC * _NS
_EW = _E // _NW
_C = 40
_NCHUNK = _EW // _C
_RPT = _N // _NS

_f32 = jnp.float32
_i32 = jnp.int32


def _gauss_body(ea_ref, mu1_ref, sg1_ref, mu2_ref, sg2_ref, mus_ref,
                sgs_ref, g1_ref, g2_ref, gs_ref):
    def gm(mu_ref, sg_ref, out_ref, nk):
        for k in range(nk):
            acc = None
            for dmn in range(_DIM):
                mu = mu_ref[k:k + 1, dmn:dmn + 1]
                sg = sg_ref[k:k + 1, dmn:dmn + 1]
                inv = -0.5 / (_EPS + sg * sg)
                dif = ea_ref[dmn:dmn + 1, :] - mu
                term = dif * dif * inv
                acc = term if acc is None else acc + term
            out_ref[k:k + 1, :] = jnp.exp(acc)

    gm(mu1_ref, sg1_ref, g1_ref, _K)
    gm(mu2_ref, sg2_ref, g2_ref, _K)
    gm(mus_ref, sgs_ref, gs_ref, 1)


def _gauss_weights(edge_attr_t, mu1, sigma1, mu2, sigma2, mus, sigmas):
    eb = 16000
    grid = _E // eb
    full = lambda s: pl.BlockSpec(s, lambda i: (0, 0))
    return pl.pallas_call(
        _gauss_body,
        grid=(grid,),
        in_specs=[pl.BlockSpec((_DIM, eb), lambda i: (0, i)),
                  full((_K, _DIM)), full((_K, _DIM)),
                  full((_K, _DIM)), full((_K, _DIM)),
                  full((1, _DIM)), full((1, _DIM))],
        out_specs=[pl.BlockSpec((_K, eb), lambda i: (0, i)),
                   pl.BlockSpec((_K, eb), lambda i: (0, i)),
                   pl.BlockSpec((1, eb), lambda i: (0, i))],
        out_shape=[jax.ShapeDtypeStruct((_K, _E), _f32),
                   jax.ShapeDtypeStruct((_K, _E), _f32),
                   jax.ShapeDtypeStruct((1, _E), _f32)],
    )(edge_attr_t, mu1, sigma1, mu2, sigma2, mus, sigmas)


def _dense_body(nout, x_ref, *refs):
    x = x_ref[...]
    for wi in range(nout):
        refs[nout + wi][...] = lax.dot_general(
            x, refs[wi][...], (((1,), (0,)), ((), ())),
            preferred_element_type=_f32)


def _dense(x, weights):
    nb = 2000
    grid = x.shape[0] // nb
    din = x.shape[1]
    in_specs = [pl.BlockSpec((nb, din), lambda i: (i, 0))]
    out_specs, out_shape = [], []
    for w in weights:
        dout = w.shape[1]
        in_specs.append(pl.BlockSpec((din, dout), lambda i: (0, 0)))
        out_specs.append(pl.BlockSpec((nb, dout), lambda i: (i, 0)))
        out_shape.append(jax.ShapeDtypeStruct((x.shape[0], dout), _f32))
    return pl.pallas_call(
        functools.partial(_dense_body, len(weights)),
        grid=(grid,),
        in_specs=in_specs, out_specs=out_specs, out_shape=out_shape,
    )(x, *weights)


def _bn(y, gamma, beta):
    m = jnp.mean(y, axis=0)
    v = jnp.mean((y - m) ** 2, axis=0)
    return gamma * (y - m) / jnp.sqrt(v + 1e-5) + beta


def _elu(y):
    return jnp.where(y > 0, y, jnp.exp(jnp.minimum(y, 0.0)) - 1.0)


def _mid_body(acc_ref, accs_ref, xr_ref, b_ref, gam_ref, bet_ref, h_ref,
              cnt_ref):
    s = acc_ref[0:_N, :] + acc_ref[_N:2 * _N, :]
    c = accs_ref[0:_N, _D:_D + 1] + accs_ref[_N:2 * _N, _D:_D + 1]
    cnt = jnp.maximum(c, 1.0)
    y = s / cnt + xr_ref[...] + b_ref[...][None, :]
    h_ref[...] = _elu(_bn(y, gam_ref[...][None, :], bet_ref[...][None, :]))
    cnt_ref[...] = cnt


def _mid(acc, accs, xr, bias, gamma, beta):
    return pl.pallas_call(
        _mid_body,
        out_shape=[jax.ShapeDtypeStruct((_N, _D), _f32),
                   jax.ShapeDtypeStruct((_N, 1), _f32)],
    )(acc, accs, xr, bias, gamma, beta)


def _final_body(acc2_ref, accs_ref, cnt_ref, xr2_ref, b2_ref, gam2_ref,
                bet2_ref, xrs_ref, bs_ref, gams_ref, bets_ref, o_ref):
    cnt = cnt_ref[...]
    y2 = ((acc2_ref[0:_N, :] + acc2_ref[_N:2 * _N, :]) / cnt
          + xr2_ref[...] + b2_ref[...][None, :])
    h = _bn(y2, gam2_ref[...][None, :], bet2_ref[...][None, :])
    ys = ((accs_ref[0:_N, 0:_D] + accs_ref[_N:2 * _N, 0:_D]) / cnt
          + xrs_ref[...] + bs_ref[...][None, :])
    sc = _bn(ys, gams_ref[...][None, :], bets_ref[...][None, :])
    o_ref[...] = _elu(h + sc)


def _final(acc2, accs, cnt, xr2, b2, gam2, bet2, xrs, bs, gams, bets):
    return pl.pallas_call(
        _final_body,
        out_shape=jax.ShapeDtypeStruct((_N, _D), _f32),
        compiler_params=pltpu.CompilerParams(
            vmem_limit_bytes=100 * 1024 * 1024),
    )(acc2, accs, cnt, xr2, b2, gam2, bet2, xrs, bs, gams, bets)


def _pack_bf16(t):
    n, d = t.shape
    tb = t.astype(jnp.bfloat16).reshape(n, d // 2, 2)
    return jax.lax.bitcast_convert_type(tb, _i32)


def _sc_agg_body(k_loc, w, packed, table, srcs, dsts, gss, zeros_in, out,
                 src_v0, src_v1, dst_v0, dst_v1, gss_v0, gss_v1,
                 rows_v0, rows_v1, msg_v, acc_sh,
                 sem_i0, sem_i1, sem_r0, sem_r1):
    src_v = [src_v0, src_v1]
    dst_v = [dst_v0, dst_v1]
    gss_v = [gss_v0, gss_v1]
    rows_v = [rows_v0, rows_v1]
    sem_i = [sem_i0, sem_i1]
    sem_r = [sem_r0, sem_r1]

    cid = lax.axis_index("c")
    sid = lax.axis_index("s")
    wid = cid * _NS + sid

    r0 = sid * _RPT
    pltpu.sync_copy(zeros_in.at[pl.ds(r0, _RPT)], acc_sh.at[pl.ds(r0, _RPT)])

    iota16 = lax.iota(_i32, 16)
    if w > _D:
        ones16 = jnp.ones((16,), _f32)

        def ones_init(r, carry):
            plsc.store_scatter(msg_v, [jnp.full((16,), r, _i32), iota16 + _D],
                               ones16)
            return carry
        lax.fori_loop(0, _C, ones_init, 0)

    plsc.subcore_barrier()

    def issue_idx(j, p):
        base = wid * _EW + j * _C
        pltpu.async_copy(srcs.at[pl.ds(base, _C)], src_v[p], sem_i[p])
        pltpu.async_copy(dsts.at[pl.ds(base, _C)], dst_v[p], sem_i[p])
        pltpu.async_copy(gss.at[:, pl.ds(base, _C)], gss_v[p], sem_i[p])

    def wait_idx(p):
        pltpu.make_async_copy(srcs.at[pl.ds(0, _C)], src_v[p],
                              sem_i[p]).wait()
        pltpu.make_async_copy(dsts.at[pl.ds(0, _C)], dst_v[p],
                              sem_i[p]).wait()
        pltpu.make_async_copy(gss.at[:, pl.ds(0, _C)], gss_v[p],
                              sem_i[p]).wait()

    def issue_rows(p):
        pltpu.async_copy(table.at[src_v[p]], rows_v[p], sem_r[p])

    def wait_rows(p):
        pltpu.make_async_copy(table.at[src_v[p]], rows_v[p],
                              sem_r[p]).wait()

    if packed:
        st_cols = ([iota16 * 2 + 32 * b for b in range(_D // 32)]
                   + [iota16 * 2 + 1 + 32 * b for b in range(_D // 32)])
    else:
        msg_cols = [iota16 + (j8 * 16) for j8 in range(_D // 16)]

    def compute_chunk(p):
        def do_edge(e):
            row_ix = jnp.full((16,), e, _i32)
            e_ix = jnp.full((16,), e, _i32)
            gb = [plsc.load_gather(
                      gss_v[p], [jnp.full((16,), k, _i32), e_ix])
                  for k in range(k_loc)]
            if packed:
                nb = _D // 32
                acc_e = [None] * nb
                acc_o = [None] * nb
                for k in range(k_loc):
                    for b in range(nb):
                        col = iota16 + (k * (_D // 2) + b * 16)
                        wv = plsc.load_gather(rows_v[p], [row_ix, col])
                        lo, hi = plsc.unpack(
                            plsc.bitcast(wv, jnp.bfloat16),
                            format=plsc.PackFormat.INTERLEAVED)
                        if k == 0:
                            acc_e[b] = gb[0] * lo
                            acc_o[b] = gb[0] * hi
                        else:
                            acc_e[b] = acc_e[b] + gb[k] * lo
                            acc_o[b] = acc_o[b] + gb[k] * hi
                for b in range(nb):
                    plsc.store_scatter(msg_v, [row_ix, st_cols[b]], acc_e[b])
                    plsc.store_scatter(msg_v, [row_ix, st_cols[nb + b]],
                                       acc_o[b])
            else:
                for j8 in range(_D // 16):
                    acc = gb[0] * plsc.load_gather(
                        rows_v[p], [row_ix, msg_cols[j8]])
                    for k in range(1, k_loc):
                        acc = acc + gb[k] * plsc.load_gather(
                            rows_v[p], [row_ix, msg_cols[j8] + k * _D])
                    plsc.store_scatter(msg_v, [row_ix, msg_cols[j8]], acc)

        def group(g, c2):
            for el in range(4):
                do_edge(g * 4 + el)
            return c2
        lax.fori_loop(0, _C // 4, group, 0)

    issue_idx(0, 0)
    issue_idx(1, 1)
    wait_idx(0)
    issue_rows(0)

    def pair(t, carry):
        for p in range(2):
            j = t * 2 + p
            q = 1 - p
            wait_rows(p)
            wait_idx(q)
            issue_rows(q)
            compute_chunk(p)
            pltpu.sync_copy(msg_v, acc_sh.at[dst_v[p]], add=True)
            issue_idx(lax.rem(j + 2, _NCHUNK), p)
        return carry

    lax.fori_loop(0, _NCHUNK // 2, pair, 0)
    wait_rows(0)
    wait_idx(1)
    plsc.subcore_barrier()

    pltpu.sync_copy(acc_sh.at[pl.ds(r0, _RPT)],
                    out.at[pl.ds(cid * _N + r0, _RPT)])


def _sc_agg(table, srcs, dsts, gss_flat, with_cnt):
    packed = table.dtype == _i32
    k_loc = table.shape[1] // (_D // 2 if packed else _D)
    w = _D + 16 if with_cnt else _D
    tw = table.shape[1]
    zeros_in = jnp.zeros((_N, w), _f32)
    tdt = _i32 if packed else _f32
    mesh = plsc.VectorSubcoreMesh(core_axis_name="c", subcore_axis_name="s")
    kfn = pl.kernel(
        functools.partial(_sc_agg_body, k_loc, w, packed),
        out_type=jax.ShapeDtypeStruct((2 * _N, w), _f32),
        mesh=mesh,
        compiler_params=pltpu.CompilerParams(use_tc_tiling_on_sc=False,
                                             needs_layout_passes=False),
        scratch_types=[
            pltpu.VMEM((_C,), _i32),
            pltpu.VMEM((_C,), _i32),
            pltpu.VMEM((_C,), _i32),
            pltpu.VMEM((_C,), _i32),
            pltpu.VMEM((k_loc, _C), _f32),
            pltpu.VMEM((k_loc, _C), _f32),
            pltpu.VMEM((_C, tw), tdt),
            pltpu.VMEM((_C, tw), tdt),
            pltpu.VMEM((_C, w), _f32),
            pltpu.VMEM_SHARED((_N, w), _f32),
            pltpu.SemaphoreType.DMA,
            pltpu.SemaphoreType.DMA,
            pltpu.SemaphoreType.DMA,
            pltpu.SemaphoreType.DMA,
        ],
    )
    return kfn(table, srcs, dsts, gss_flat, zeros_in)


def kernel(x, edge_index, edge_attr, g1, mu1, sigma1, root1, bias1, gamma1,
           beta1, g2, mu2, sigma2, root2, bias2, gamma2, beta2, gs, mus,
           sigmas, roots, biass, gammas, betas):
    src = edge_index[0]
    dst = edge_index[1]

    gauss1, gauss2, gausss = _gauss_weights(
        edge_attr.T, mu1, sigma1, mu2, sigma2, mus, sigmas)

    xg1, xr1, xgs, xrs = _dense(x, [g1, root1, gs, roots])
    xg1p = _pack_bf16(xg1)

    accs = _sc_agg(xgs, src, dst, gausss, with_cnt=True)
    acc1 = _sc_agg(xg1p, src, dst, gauss1, with_cnt=False)
    h, cnt = _mid(acc1, accs, xr1, bias1, gamma1, beta1)

    xg2, xr2 = _dense(h, [g2, root2])
    acc2 = _sc_agg(_pack_bf16(xg2), src, dst, gauss2, with_cnt=False)

    return _final(acc2, accs, cnt, xr2, bias2, gamma2, beta2,
                  xrs, biass, gammas, betas)
